# baseline (speedup 1.0000x reference)
import functools

import jax
import jax.numpy as jnp
import numpy as np
from jax import lax
from jax.experimental import pallas as pl
from jax.experimental.pallas import tpu as pltpu, tpu_sc as plsc

_IN_F = 256
_OUT_F = 64
_P = 32
_WIDTH = 4.0

_B_SC = 2048
_NW = 32
_BPW = _B_SC // _NW
_FB = 16
_NFB = _IN_F // _FB
_NJ = _OUT_F // 16


_TROWS = _FB * _P * 32 // 128


def _decode(v):
    lo = jax.lax.bitcast_convert_type(v << 16, jnp.float32)
    hi = jax.lax.bitcast_convert_type(v & jnp.int32(-65536), jnp.float32)
    return lo, hi


def _sc_body(x_hbm, tabi_hbm, out_hbm, xblk, tab0, tab1, acc, sem0, sem1):
    wid = lax.axis_index("s") * 2 + lax.axis_index("c")
    base = wid * _BPW
    pltpu.sync_copy(x_hbm.at[pl.ds(base, _BPW), :], xblk)

    def zero_body(b, carry):
        z = jnp.zeros((16,), jnp.float32)
        for j in range(_NJ):
            acc[b, pl.ds(j * 16, 16)] = z
        return carry

    lax.fori_loop(0, _BPW, zero_body, 0)

    pltpu.async_copy(tabi_hbm.at[pl.ds(0, _TROWS), :], tab0, sem0)

    def do_block(fb, tab):
        f0 = fb * _FB

        def row_body(b, c2):
            xs = (xblk[b, pl.ds(f0, _FB)] + _WIDTH / 2.0) * ((_P - 1) / _WIDTH)
            lo = jnp.minimum(jnp.maximum(xs, 0.0).astype(jnp.int32), _P - 2)
            t = xs - lo.astype(jnp.float32)
            a = [acc[b, pl.ds(j * 16, 16)] for j in range(_NJ)]
            for f in range(_FB):
                lo_s = lo[f]
                t_s = t[f]
                row = f * _P + lo_s
                rmaj = row >> 2
                rmin = (row & 3) * 32
                v0 = tab[rmaj, pl.ds(rmin, 16)]
                v1 = tab[rmaj, pl.ds(rmin + 16, 16)]
                nmaj = rmaj + (rmin + 32) // 128
                nmin = (rmin + 32) % 128
                w0 = tab[nmaj, pl.ds(nmin, 16)]
                w1 = tab[nmaj, pl.ds(nmin + 16, 16)]
                rl0, rl1 = _decode(v0)
                rl2, rl3 = _decode(v1)
                rh0, rh1 = _decode(w0)
                rh2, rh3 = _decode(w1)
                rls = (rl0, rl1, rl2, rl3)
                rhs = (rh0, rh1, rh2, rh3)
                for j in range(_NJ):
                    a[j] = a[j] + rls[j] + t_s * (rhs[j] - rls[j])
            for j in range(_NJ):
                acc[b, pl.ds(j * 16, 16)] = a[j]
            return c2

        lax.fori_loop(0, _BPW, row_body, 0)

    def pair_body(pair, carry):
        fb0 = 2 * pair
        fb1 = 2 * pair + 1
        pltpu.async_copy(tabi_hbm.at[pl.ds(fb1 * _TROWS, _TROWS), :], tab1, sem1)
        pltpu.make_async_copy(
            tabi_hbm.at[pl.ds(fb0 * _TROWS, _TROWS), :], tab0, sem0
        ).wait()
        do_block(fb0, tab0)

        @pl.when(pair < _NFB // 2 - 1)
        def _():
            pltpu.async_copy(
                tabi_hbm.at[pl.ds((fb0 + 2) * _TROWS, _TROWS), :], tab0, sem0
            )

        pltpu.make_async_copy(
            tabi_hbm.at[pl.ds(fb1 * _TROWS, _TROWS), :], tab1, sem1
        ).wait()
        do_block(fb1, tab1)
        return carry

    lax.fori_loop(0, _NFB // 2, pair_body, 0)
    pltpu.sync_copy(acc, out_hbm.at[pl.ds(base, _BPW), :])


def _sc_part(x, tabi):
    mesh = plsc.VectorSubcoreMesh(core_axis_name="c", subcore_axis_name="s")
    f = functools.partial(
        pl.kernel,
        mesh=mesh,
        out_type=jax.ShapeDtypeStruct((_B_SC, _OUT_F), jnp.float32),
        scratch_types=[
            pltpu.VMEM((_BPW, _IN_F), jnp.float32),
            pltpu.VMEM((_TROWS, 128), jnp.int32),
            pltpu.VMEM((_TROWS, 128), jnp.int32),
            pltpu.VMEM((_BPW, _OUT_F), jnp.float32),
            pltpu.SemaphoreType.DMA,
            pltpu.SemaphoreType.DMA,
        ],
    )(_sc_body)
    return f(x, tabi)


def _pack_table(tab):
    u16 = jax.lax.bitcast_convert_type(tab.astype(jnp.bfloat16), jnp.uint16)
    u16r = u16.reshape(_IN_F * _P // 4, 256)
    c = np.arange(128)
    s, m0 = c // 32, c % 32
    perm_lo = s * 64 + np.where(m0 < 16, m0, m0 + 16)
    lo = u16r[:, perm_lo].astype(jnp.uint32)
    hi = u16r[:, perm_lo + 16].astype(jnp.uint32)
    return jax.lax.bitcast_convert_type(lo | (hi << 16), jnp.int32)


def _tc_body(x_ref, vt_ref, b_ref, o_ref):
    xs = (x_ref[...] + _WIDTH / 2.0) * ((_P - 1) / _WIDTH)
    dot = lambda a, b: jax.lax.dot(a, b, preferred_element_type=jnp.float32)
    acc = b_ref[0:1, :] + dot(xs, vt_ref[:, : _OUT_F])
    for k in range(1, _P - 1):
        acc += dot(
            jnp.maximum(xs - float(k), 0.0),
            vt_ref[:, k * _OUT_F : (k + 1) * _OUT_F],
        )
    o_ref[...] = acc


def _tc_part(x, vtab, bias, bt):
    b = x.shape[0]
    skip = _B_SC // bt
    return pl.pallas_call(
        _tc_body,
        grid=(b // bt - skip,),
        in_specs=[
            pl.BlockSpec((bt, _IN_F), lambda i: (i + skip, 0)),
            pl.BlockSpec((_IN_F, _P * _OUT_F), lambda i: (0, 0)),
            pl.BlockSpec((8, _OUT_F), lambda i: (0, 0)),
        ],
        out_specs=pl.BlockSpec((bt, _OUT_F), lambda i: (i + skip, 0)),
        out_shape=jax.ShapeDtypeStruct((b, _OUT_F), jnp.float32),
    )(x, vtab, bias)


def _knot_tables(kan_weight):
    kw2 = kan_weight.reshape(_IN_F, _P * _OUT_F)
    z = jnp.zeros((_IN_F, _OUT_F), jnp.float32)
    up = jnp.concatenate([kw2[:, _OUT_F:], z], axis=1)
    dn = jnp.concatenate([z, kw2[:, : -_OUT_F]], axis=1)
    g = jnp.arange(_P * _OUT_F, dtype=jnp.int32) // _OUT_F
    ca = jnp.where(g <= _P - 2, 1.0, 0.0)[None, :]
    cb = jnp.where(g == 0, -1.0, jnp.where(g <= _P - 2, -2.0, 0.0))[None, :]
    cc = jnp.where((g >= 1) & (g <= _P - 2), 1.0, 0.0)[None, :]
    vtab = up * ca + kw2 * cb + dn * cc
    bias = jnp.broadcast_to(
        jnp.sum(kan_weight[:, 0, :], axis=0)[None, :], (8, _OUT_F)
    )
    return vtab, bias


def kernel(x, kan_weight):
    tabi = _pack_table(kan_weight.reshape(_IN_F * _P, _OUT_F))
    vtab, bias = _knot_tables(kan_weight)
    out_sc = _sc_part(x, tabi)
    out_tc = _tc_part(x, vtab, bias, bt=512)
    return lax.dynamic_update_slice(out_tc, out_sc, (0, 0))


# speedup vs baseline: 1.3114x; 1.3114x over previous
import functools

import jax
import jax.numpy as jnp
import numpy as np
from jax import lax
from jax.experimental import pallas as pl
from jax.experimental.pallas import tpu as pltpu, tpu_sc as plsc

_IN_F = 256
_OUT_F = 64
_P = 32
_WIDTH = 4.0

_B_SC = 1024
_NW = 32
_BPW = _B_SC // _NW
_FB = 16
_NFB = _IN_F // _FB
_NJ = _OUT_F // 16


_TROWS = _FB * _P * 32 // 128


def _decode(v):
    lo = jax.lax.bitcast_convert_type(v << 16, jnp.float32)
    hi = jax.lax.bitcast_convert_type(v & jnp.int32(-65536), jnp.float32)
    return lo, hi


def _sc_body(x_hbm, tabi_hbm, out_hbm, xblk, tab0, tab1, acc, sem0, sem1):
    wid = lax.axis_index("s") * 2 + lax.axis_index("c")
    base = wid * _BPW
    pltpu.sync_copy(x_hbm.at[pl.ds(base, _BPW), :], xblk)

    def zero_body(b, carry):
        z = jnp.zeros((16,), jnp.float32)
        for j in range(_NJ):
            acc[b, pl.ds(j * 16, 16)] = z
        return carry

    lax.fori_loop(0, _BPW, zero_body, 0)

    pltpu.async_copy(tabi_hbm.at[pl.ds(0, _TROWS), :], tab0, sem0)

    def do_block(fb, tab):
        f0 = fb * _FB

        def row_body(b, c2):
            xs = (xblk[b, pl.ds(f0, _FB)] + _WIDTH / 2.0) * ((_P - 1) / _WIDTH)
            lo = jnp.minimum(jnp.maximum(xs, 0.0).astype(jnp.int32), _P - 2)
            t = xs - lo.astype(jnp.float32)
            a = [acc[b, pl.ds(j * 16, 16)] for j in range(_NJ)]
            for f in range(_FB):
                lo_s = lo[f]
                t_s = t[f]
                row = f * _P + lo_s
                rmaj = row >> 2
                rmin = (row & 3) * 32
                v0 = tab[rmaj, pl.ds(rmin, 16)]
                v1 = tab[rmaj, pl.ds(rmin + 16, 16)]
                nmaj = rmaj + (rmin + 32) // 128
                nmin = (rmin + 32) % 128
                w0 = tab[nmaj, pl.ds(nmin, 16)]
                w1 = tab[nmaj, pl.ds(nmin + 16, 16)]
                rl0, rl1 = _decode(v0)
                rl2, rl3 = _decode(v1)
                rh0, rh1 = _decode(w0)
                rh2, rh3 = _decode(w1)
                rls = (rl0, rl1, rl2, rl3)
                rhs = (rh0, rh1, rh2, rh3)
                for j in range(_NJ):
                    a[j] = a[j] + rls[j] + t_s * (rhs[j] - rls[j])
            for j in range(_NJ):
                acc[b, pl.ds(j * 16, 16)] = a[j]
            return c2

        lax.fori_loop(0, _BPW, row_body, 0)

    def pair_body(pair, carry):
        fb0 = 2 * pair
        fb1 = 2 * pair + 1
        pltpu.async_copy(tabi_hbm.at[pl.ds(fb1 * _TROWS, _TROWS), :], tab1, sem1)
        pltpu.make_async_copy(
            tabi_hbm.at[pl.ds(fb0 * _TROWS, _TROWS), :], tab0, sem0
        ).wait()
        do_block(fb0, tab0)

        @pl.when(pair < _NFB // 2 - 1)
        def _():
            pltpu.async_copy(
                tabi_hbm.at[pl.ds((fb0 + 2) * _TROWS, _TROWS), :], tab0, sem0
            )

        pltpu.make_async_copy(
            tabi_hbm.at[pl.ds(fb1 * _TROWS, _TROWS), :], tab1, sem1
        ).wait()
        do_block(fb1, tab1)
        return carry

    lax.fori_loop(0, _NFB // 2, pair_body, 0)
    pltpu.sync_copy(acc, out_hbm.at[pl.ds(base, _BPW), :])


def _sc_part(x, tabi):
    mesh = plsc.VectorSubcoreMesh(core_axis_name="c", subcore_axis_name="s")
    f = functools.partial(
        pl.kernel,
        mesh=mesh,
        out_type=jax.ShapeDtypeStruct((_B_SC, _OUT_F), jnp.float32),
        scratch_types=[
            pltpu.VMEM((_BPW, _IN_F), jnp.float32),
            pltpu.VMEM((_TROWS, 128), jnp.int32),
            pltpu.VMEM((_TROWS, 128), jnp.int32),
            pltpu.VMEM((_BPW, _OUT_F), jnp.float32),
            pltpu.SemaphoreType.DMA,
            pltpu.SemaphoreType.DMA,
        ],
    )(_sc_body)
    return f(x, tabi)


def _pack_table(tab):
    u16 = jax.lax.bitcast_convert_type(tab.astype(jnp.bfloat16), jnp.uint16)
    u16r = u16.reshape(_IN_F * _P // 4, 256)
    c = np.arange(128)
    s, m0 = c // 32, c % 32
    perm_lo = s * 64 + np.where(m0 < 16, m0, m0 + 16)
    lo = u16r[:, perm_lo].astype(jnp.uint32)
    hi = u16r[:, perm_lo + 16].astype(jnp.uint32)
    return jax.lax.bitcast_convert_type(lo | (hi << 16), jnp.int32)


def _tc_body(x_ref, vt_ref, b_ref, o_ref):
    xs = (x_ref[...] + _WIDTH / 2.0) * ((_P - 1) / _WIDTH)
    dot = lambda a, b: jax.lax.dot(a, b, preferred_element_type=jnp.float32)
    acc = b_ref[0:1, :] + dot(xs, vt_ref[:, : _OUT_F])
    for k in range(1, _P - 1):
        acc += dot(
            jnp.maximum(xs - float(k), 0.0),
            vt_ref[:, k * _OUT_F : (k + 1) * _OUT_F],
        )
    o_ref[...] = acc


def _tc_part(x, vtab, bias, bt):
    b = x.shape[0]
    skip = _B_SC // bt
    return pl.pallas_call(
        _tc_body,
        grid=(b // bt - skip,),
        in_specs=[
            pl.BlockSpec((bt, _IN_F), lambda i: (i + skip, 0)),
            pl.BlockSpec((_IN_F, _P * _OUT_F), lambda i: (0, 0)),
            pl.BlockSpec((8, _OUT_F), lambda i: (0, 0)),
        ],
        out_specs=pl.BlockSpec((bt, _OUT_F), lambda i: (i + skip, 0)),
        out_shape=jax.ShapeDtypeStruct((b, _OUT_F), jnp.float32),
    )(x, vtab, bias)


def _knot_tables(kan_weight):
    kw2 = kan_weight.reshape(_IN_F, _P * _OUT_F)
    z = jnp.zeros((_IN_F, _OUT_F), jnp.float32)
    up = jnp.concatenate([kw2[:, _OUT_F:], z], axis=1)
    dn = jnp.concatenate([z, kw2[:, : -_OUT_F]], axis=1)
    g = jnp.arange(_P * _OUT_F, dtype=jnp.int32) // _OUT_F
    ca = jnp.where(g <= _P - 2, 1.0, 0.0)[None, :]
    cb = jnp.where(g == 0, -1.0, jnp.where(g <= _P - 2, -2.0, 0.0))[None, :]
    cc = jnp.where((g >= 1) & (g <= _P - 2), 1.0, 0.0)[None, :]
    vtab = up * ca + kw2 * cb + dn * cc
    bias = jnp.broadcast_to(
        jnp.sum(kan_weight[:, 0, :], axis=0)[None, :], (8, _OUT_F)
    )
    return vtab, bias


def kernel(x, kan_weight):
    tabi = _pack_table(kan_weight.reshape(_IN_F * _P, _OUT_F))
    vtab, bias = _knot_tables(kan_weight)
    out_sc = _sc_part(x, tabi)
    out_tc = _tc_part(x, vtab, bias, bt=512)
    return lax.dynamic_update_slice(out_tc, out_sc, (0, 0))
